# Initial kernel scaffold; baseline (speedup 1.0000x reference)
import jax
import jax.numpy as jnp
from jax.experimental import pallas as pl


def kernel(x, edge_index, e, u, batch, W_e1, b_e1, W_e2, b_e2, W_a1, b_a1, W_a2, b_a2, W_n1, b_n1, W_n2, b_n2, W_g1, b_g1, W_g2, b_g2):
    raise NotImplementedError("write your pallas kernel here")


# speedup vs baseline: 6.1536x; 6.1536x over previous
import functools

import jax
import jax.numpy as jnp
from jax import lax
from jax.experimental import pallas as pl
from jax.experimental.pallas import tpu as pltpu
from jax.experimental.pallas import tpu_sc as plsc

NC = 2
NS = 16
NW = NC * NS
CH = 80
KSUB = 5
GRP = CH * KSUB


def _mesh():
    return plsc.VectorSubcoreMesh(
        core_axis_name="c", subcore_axis_name="s",
        num_cores=NC, num_subcores=NS)


_SC_PARAMS = pltpu.CompilerParams(use_tc_tiling_on_sc=False)


def _gather_body(groups_per_tile, ws, wd,
                 asrc_hbm, adst_hbm, src3_hbm, dst3_hbm,
                 outs_hbm, outd_hbm,
                 sidx, didx, gs, gd, sem_s, sem_d):
    wid = lax.axis_index("c") * NS + lax.axis_index("s")
    grp0 = wid * groups_per_tile
    base0 = wid * (groups_per_tile * GRP)

    def body(g, carry):
        b = base0 + g * GRP
        pltpu.sync_copy(src3_hbm.at[grp0 + g], sidx)
        pltpu.sync_copy(dst3_hbm.at[grp0 + g], didx)
        cps = []
        for j in range(KSUB):
            cps.append(pltpu.async_copy(
                asrc_hbm.at[sidx.at[j]], gs.at[pl.ds(j * CH, CH)], sem_s))
            cps.append(pltpu.async_copy(
                adst_hbm.at[didx.at[j]], gd.at[pl.ds(j * CH, CH)], sem_d))
        for c in cps:
            c.wait()
        pltpu.sync_copy(gs, outs_hbm.at[pl.ds(b, GRP)])
        pltpu.sync_copy(gd, outd_hbm.at[pl.ds(b, GRP)])
        return carry

    lax.fori_loop(0, groups_per_tile, body, 0)


def _make_gather(E, ws, wd):
    ept = E // NW
    groups_per_tile = ept // GRP
    return functools.partial(
        pl.kernel,
        out_type=[jax.ShapeDtypeStruct((E, ws), jnp.float32),
                  jax.ShapeDtypeStruct((E, wd), jnp.float32)],
        mesh=_mesh(),
        scratch_types=[
            pltpu.VMEM((KSUB, CH), jnp.int32),
            pltpu.VMEM((KSUB, CH), jnp.int32),
            pltpu.VMEM((GRP, ws), jnp.float32),
            pltpu.VMEM((GRP, wd), jnp.float32),
            pltpu.SemaphoreType.DMA,
            pltpu.SemaphoreType.DMA,
        ],
        compiler_params=_SC_PARAMS,
    )(functools.partial(_gather_body, groups_per_tile, ws, wd))


def _scatter_body(groups_per_tile, npad, de,
                  msg_hbm, dst3_hbm, zeros_hbm, out_hbm,
                  midx, mv, table):
    cid = lax.axis_index("c")
    sid = lax.axis_index("s")
    wid = cid * NS + sid
    rps = npad // NS
    pltpu.sync_copy(zeros_hbm, table.at[pl.ds(sid * rps, rps)])
    plsc.subcore_barrier()

    grp0 = wid * groups_per_tile
    base0 = wid * (groups_per_tile * GRP)

    def body(g, carry):
        b = base0 + g * GRP
        pltpu.sync_copy(dst3_hbm.at[grp0 + g], midx)
        pltpu.sync_copy(msg_hbm.at[pl.ds(b, GRP)], mv)
        for j in range(KSUB):
            pltpu.sync_copy(mv.at[pl.ds(j * CH, CH)],
                            table.at[midx.at[j]], add=True)
        return carry

    lax.fori_loop(0, groups_per_tile, body, 0)
    plsc.subcore_barrier()
    pltpu.sync_copy(table.at[pl.ds(sid * rps, rps)],
                    out_hbm.at[cid, pl.ds(sid * rps, rps)])


def _make_scatter(E, npad, de):
    ept = E // NW
    groups_per_tile = ept // GRP
    return functools.partial(
        pl.kernel,
        out_type=[jax.ShapeDtypeStruct((NC, npad, de), jnp.float32)],
        mesh=_mesh(),
        scratch_types=[
            pltpu.VMEM((KSUB, CH), jnp.int32),
            pltpu.VMEM((GRP, de), jnp.float32),
            pltpu.VMEM_SHARED((npad, de), jnp.float32),
        ],
        compiler_params=_SC_PARAMS,
    )(functools.partial(_scatter_body, groups_per_tile, npad, de))


def _prep_body(x_ref, oh_ref, u_ref, wes_ref, was_ref, wed_ref, wad_ref,
               weu_ref, wau_ref, be1_ref, ba1_ref, outs_ref, outd_ref):
    f32 = jnp.float32
    xb = x_ref[...]
    oh = oh_ref[...]
    u = u_ref[...]
    upe = jnp.dot(u, weu_ref[...], preferred_element_type=f32) + be1_ref[...]
    upa = jnp.dot(u, wau_ref[...], preferred_element_type=f32) + ba1_ref[...]
    pse = (jnp.dot(xb, wes_ref[...], preferred_element_type=f32)
           + jnp.dot(oh, upe, preferred_element_type=f32))
    psa = (jnp.dot(xb, was_ref[...], preferred_element_type=f32)
           + jnp.dot(oh, upa, preferred_element_type=f32))
    pad = jnp.zeros(oh.shape, f32)
    outs_ref[...] = jnp.concatenate([pse, psa, oh, pad], axis=1)
    outd_ref[...] = jnp.concatenate(
        [jnp.dot(xb, wed_ref[...], preferred_element_type=f32),
         jnp.dot(xb, wad_ref[...], preferred_element_type=f32)], axis=1)


def _edge_body(nblk, gs_ref, gd_ref, e_ref, we1e_ref, we2_ref, be2_ref,
               wa1e_ref, wa2_ref, ba2_ref,
               enew_ref, msg_ref, esum_ref, ecnt_ref):
    f32 = jnp.float32
    gs = gs_ref[...]
    gd = gd_ref[...]
    h = we1e_ref.shape[1]
    de = we2_ref.shape[1]
    ze = (gs[:, 0:h] + gd[:, 0:h]
          + jnp.dot(e_ref[...], we1e_ref[...], preferred_element_type=f32))
    he = jnp.maximum(ze, 0.0)
    en = jnp.dot(he, we2_ref[...], preferred_element_type=f32) + be2_ref[...]
    za = (gs[:, h:2 * h] + gd[:, h:2 * h]
          + jnp.dot(en, wa1e_ref[...], preferred_element_type=f32))
    ha = jnp.maximum(za, 0.0)
    aa = jnp.dot(ha, wa2_ref[...], preferred_element_type=f32) + ba2_ref[...]
    a = 1.0 / (1.0 + jnp.exp(-aa))
    enew_ref[...] = en
    msg_ref[...] = en * a
    B = ecnt_ref.shape[0]
    oh = gs[:, 2 * h:2 * h + B]
    es = lax.dot_general(oh, en, (((0,), (0,)), ((), ())),
                         preferred_element_type=f32)
    ec = jnp.broadcast_to(jnp.sum(oh, axis=0)[:, None], (B, de))

    @pl.when(pl.program_id(0) == 0)
    def _():
        esum_ref[...] = jnp.zeros_like(esum_ref)
        ecnt_ref[...] = jnp.zeros_like(ecnt_ref)

    esum_ref[...] += es
    ecnt_ref[...] += ec


def _final_body(nblk, x_ref, a0_ref, a1_ref, oh_ref, u_ref, esum_ref,
                ecnt_ref, wn1x_ref, wn1a_ref, wn1u_ref, bn1_ref, wn2_ref,
                bn2_ref, wg1n_ref, wg1e_ref, wg1u_ref, bg1_ref, wg2_ref,
                bg2_ref, xnew_ref, unew_ref, nsum_scr, ncnt_scr):
    f32 = jnp.float32
    i = pl.program_id(0)
    oh = oh_ref[...]
    agg = a0_ref[...] + a1_ref[...]
    xb = x_ref[...]
    u = u_ref[...]
    un1 = jnp.dot(u, wn1u_ref[...], preferred_element_type=f32)
    hn = jnp.maximum(
        jnp.dot(xb, wn1x_ref[...], preferred_element_type=f32)
        + jnp.dot(agg, wn1a_ref[...], preferred_element_type=f32)
        + jnp.dot(oh, un1, preferred_element_type=f32) + bn1_ref[...], 0.0)
    xn = jnp.dot(hn, wn2_ref[...], preferred_element_type=f32) + bn2_ref[...]
    xnew_ref[...] = xn

    @pl.when(i == 0)
    def _():
        nsum_scr[...] = jnp.zeros_like(nsum_scr)
        ncnt_scr[...] = jnp.zeros_like(ncnt_scr)

    d = nsum_scr.shape[1]
    B = nsum_scr.shape[0]
    nsum_scr[...] += lax.dot_general(oh, xn, (((0,), (0,)), ((), ())),
                                     preferred_element_type=f32)
    ncnt_scr[...] += jnp.broadcast_to(jnp.sum(oh, axis=0)[:, None], (B, d))

    @pl.when(i == nblk - 1)
    def _():
        nm = nsum_scr[...] / jnp.maximum(ncnt_scr[...], 1.0)
        em = esum_ref[...] / jnp.maximum(ecnt_ref[...], 1.0)
        hg = jnp.maximum(
            jnp.dot(nm, wg1n_ref[...], preferred_element_type=f32)
            + jnp.dot(em, wg1e_ref[...], preferred_element_type=f32)
            + jnp.dot(u, wg1u_ref[...], preferred_element_type=f32)
            + bg1_ref[...], 0.0)
        unew_ref[...] = (jnp.dot(hg, wg2_ref[...], preferred_element_type=f32)
                         + bg2_ref[...])


def kernel(x, edge_index, e, u, batch,
           W_e1, b_e1, W_e2, b_e2,
           W_a1, b_a1, W_a2, b_a2,
           W_n1, b_n1, W_n2, b_n2,
           W_g1, b_g1, W_g2, b_g2):
    f32 = jnp.float32
    N, d = x.shape
    E, de = e.shape
    B, du = u.shape
    h = W_e1.shape[1]

    src3 = edge_index[0].reshape(E // GRP, KSUB, CH)
    dst3 = edge_index[1].reshape(E // GRP, KSUB, CH)
    oh = (batch[:, None] == jnp.arange(B, dtype=batch.dtype)[None, :]
          ).astype(f32)
    npad = ((N + 8 * NS - 1) // (8 * NS)) * (8 * NS)
    zeros_tbl = jnp.zeros((npad // NS, de), f32)

    we1_s, we1_d, we1_e, we1_u = (W_e1[0:d], W_e1[d:2 * d],
                                  W_e1[2 * d:2 * d + de], W_e1[2 * d + de:])
    wa1_s, wa1_d, wa1_e, wa1_u = (W_a1[0:d], W_a1[d:2 * d],
                                  W_a1[2 * d:2 * d + de], W_a1[2 * d + de:])
    wn1_x, wn1_a, wn1_u = W_n1[0:d], W_n1[d:d + de], W_n1[d + de:]
    wg1_n, wg1_e, wg1_u = W_g1[0:d], W_g1[d:d + de], W_g1[d + de:]
    r2 = lambda b: b.reshape(1, -1)

    ws = 2 * h + 2 * B
    wd = 2 * h

    BN = 2000
    npb = N // BN
    wspec = lambda: pl.BlockSpec((None,), lambda i: (0,))
    prep = pl.pallas_call(
        _prep_body,
        grid=(npb,),
        in_specs=[
            pl.BlockSpec((BN, d), lambda i: (i, 0)),
            pl.BlockSpec((BN, B), lambda i: (i, 0)),
            pl.BlockSpec((B, du), lambda i: (0, 0)),
            pl.BlockSpec((d, h), lambda i: (0, 0)),
            pl.BlockSpec((d, h), lambda i: (0, 0)),
            pl.BlockSpec((d, h), lambda i: (0, 0)),
            pl.BlockSpec((d, h), lambda i: (0, 0)),
            pl.BlockSpec((du, h), lambda i: (0, 0)),
            pl.BlockSpec((du, h), lambda i: (0, 0)),
            pl.BlockSpec((1, h), lambda i: (0, 0)),
            pl.BlockSpec((1, h), lambda i: (0, 0)),
        ],
        out_specs=[
            pl.BlockSpec((BN, ws), lambda i: (i, 0)),
            pl.BlockSpec((BN, wd), lambda i: (i, 0)),
        ],
        out_shape=[jax.ShapeDtypeStruct((N, ws), f32),
                   jax.ShapeDtypeStruct((N, wd), f32)],
    )
    asrc, adst = prep(x, oh, u, we1_s, wa1_s, we1_d, wa1_d,
                      we1_u, wa1_u, r2(b_e1), r2(b_a1))

    gs, gd = _make_gather(E, ws, wd)(asrc, adst, src3, dst3)

    BE = 6400
    neb = E // BE
    edge = pl.pallas_call(
        functools.partial(_edge_body, neb),
        grid=(neb,),
        in_specs=[
            pl.BlockSpec((BE, ws), lambda i: (i, 0)),
            pl.BlockSpec((BE, wd), lambda i: (i, 0)),
            pl.BlockSpec((BE, de), lambda i: (i, 0)),
            pl.BlockSpec((de, h), lambda i: (0, 0)),
            pl.BlockSpec((h, de), lambda i: (0, 0)),
            pl.BlockSpec((1, de), lambda i: (0, 0)),
            pl.BlockSpec((de, h), lambda i: (0, 0)),
            pl.BlockSpec((h, de), lambda i: (0, 0)),
            pl.BlockSpec((1, de), lambda i: (0, 0)),
        ],
        out_specs=[
            pl.BlockSpec((BE, de), lambda i: (i, 0)),
            pl.BlockSpec((BE, de), lambda i: (i, 0)),
            pl.BlockSpec((B, de), lambda i: (0, 0)),
            pl.BlockSpec((B, de), lambda i: (0, 0)),
        ],
        out_shape=[jax.ShapeDtypeStruct((E, de), f32),
                   jax.ShapeDtypeStruct((E, de), f32),
                   jax.ShapeDtypeStruct((B, de), f32),
                   jax.ShapeDtypeStruct((B, de), f32)],
    )
    e_new, msg, esum, ecnt = edge(gs, gd, e, we1_e, W_e2, r2(b_e2),
                                  wa1_e, W_a2, r2(b_a2))

    aggp, = _make_scatter(E, npad, de)(msg, dst3, zeros_tbl)

    final = pl.pallas_call(
        functools.partial(_final_body, npb),
        grid=(npb,),
        in_specs=[
            pl.BlockSpec((BN, d), lambda i: (i, 0)),
            pl.BlockSpec((BN, de), lambda i: (i, 0)),
            pl.BlockSpec((BN, de), lambda i: (i, 0)),
            pl.BlockSpec((BN, B), lambda i: (i, 0)),
            pl.BlockSpec((B, du), lambda i: (0, 0)),
            pl.BlockSpec((B, de), lambda i: (0, 0)),
            pl.BlockSpec((B, de), lambda i: (0, 0)),
            pl.BlockSpec((d, h), lambda i: (0, 0)),
            pl.BlockSpec((de, h), lambda i: (0, 0)),
            pl.BlockSpec((du, h), lambda i: (0, 0)),
            pl.BlockSpec((1, h), lambda i: (0, 0)),
            pl.BlockSpec((h, d), lambda i: (0, 0)),
            pl.BlockSpec((1, d), lambda i: (0, 0)),
            pl.BlockSpec((d, h), lambda i: (0, 0)),
            pl.BlockSpec((de, h), lambda i: (0, 0)),
            pl.BlockSpec((du, h), lambda i: (0, 0)),
            pl.BlockSpec((1, h), lambda i: (0, 0)),
            pl.BlockSpec((h, du), lambda i: (0, 0)),
            pl.BlockSpec((1, du), lambda i: (0, 0)),
        ],
        out_specs=[
            pl.BlockSpec((BN, d), lambda i: (i, 0)),
            pl.BlockSpec((B, du), lambda i: (0, 0)),
        ],
        out_shape=[jax.ShapeDtypeStruct((N, d), f32),
                   jax.ShapeDtypeStruct((B, du), f32)],
        scratch_shapes=[pltpu.VMEM((B, d), f32),
                        pltpu.VMEM((B, d), f32)],
    )
    x_new, u_new = final(x, aggp[0, :N], aggp[1, :N], oh, u, esum, ecnt,
                         wn1_x, wn1_a, wn1_u, r2(b_n1), W_n2, r2(b_n2),
                         wg1_n, wg1_e, wg1_u, r2(b_g1), W_g2, r2(b_g2))
    return (x_new, e_new, u_new)
